# baseline (speedup 1.0000x reference)
import functools

import jax
import jax.numpy as jnp
from jax import lax
from jax.experimental import pallas as pl
from jax.experimental.pallas import tpu as pltpu
from jax.experimental.pallas import tpu_sc as plsc

N_NODES = 10000
N_EDGES = 160000
F_IN = 256
F_OUT = 256
FH = 128
NC = 2
NS = 16

CHUNK = 125
E_ROWS = N_EDGES // CHUNK
DEG_ROWS_PER_W = E_ROWS // (NC * NS)
AGG_ROWS_PER_W = E_ROWS // NS

STRIPE = 624
TAIL_ROWS = N_NODES - NS * STRIPE

_MESH = plsc.VectorSubcoreMesh(core_axis_name="c", subcore_axis_name="s")


@functools.partial(
    pl.kernel,
    out_type=[jax.ShapeDtypeStruct((N_NODES,), jnp.float32),
              jax.ShapeDtypeStruct((N_NODES,), jnp.float32)],
    mesh=_MESH,
    scratch_types=[
        pltpu.VMEM((DEG_ROWS_PER_W, CHUNK), jnp.int32),
        pltpu.VMEM((CHUNK,), jnp.float32),
        pltpu.VMEM_SHARED((N_NODES,), jnp.float32),
        pltpu.SemaphoreType.DMA,
    ],
)
def _deg_kernel(e_hbm, zeros_hbm, ones_hbm, deg0_hbm, deg1_hbm,
                idx_v, ones_v, deg_sh, sem):
    c = lax.axis_index("c")
    s = lax.axis_index("s")
    w = c * NS + s

    @pl.when(s == 0)
    def _():
        pltpu.sync_copy(zeros_hbm, deg_sh)

    pltpu.sync_copy(
        e_hbm.at[0].at[pl.ds(w * DEG_ROWS_PER_W, DEG_ROWS_PER_W)], idx_v)
    pltpu.sync_copy(ones_hbm, ones_v)
    plsc.subcore_barrier()

    @pl.loop(0, DEG_ROWS_PER_W)
    def _(j):
        pltpu.sync_copy(ones_v, deg_sh.at[idx_v.at[j]], add=True)

    plsc.subcore_barrier()

    @pl.when((s == 0) & (c == 0))
    def _():
        pltpu.sync_copy(deg_sh, deg0_hbm)

    @pl.when((s == 0) & (c == 1))
    def _():
        pltpu.sync_copy(deg_sh, deg1_hbm)


_TC_BLOCK = 2000


def _tc_body(x_ref, w_ref, deg_ref, h_ref):
    norm = lax.rsqrt(deg_ref[...] + 1.0)
    h = jnp.dot(x_ref[...].astype(jnp.bfloat16),
                w_ref[...].astype(jnp.bfloat16),
                preferred_element_type=jnp.float32)
    h = h * norm
    h_ref[0] = h[:, :FH]
    h_ref[1] = h[:, FH:]


def _tc_matmul(x, W, degsum):
    grid = (N_NODES // _TC_BLOCK,)
    return pl.pallas_call(
        _tc_body,
        grid=grid,
        in_specs=[
            pl.BlockSpec((_TC_BLOCK, F_IN), lambda i: (i, 0)),
            pl.BlockSpec((F_IN, F_OUT), lambda i: (0, 0)),
            pl.BlockSpec((_TC_BLOCK, 1), lambda i: (i, 0)),
        ],
        out_specs=pl.BlockSpec((NC, _TC_BLOCK, FH), lambda i: (0, i, 0)),
        out_shape=jax.ShapeDtypeStruct((NC, N_NODES, FH), jnp.float32),
    )(x, W, degsum)


@functools.partial(
    pl.kernel,
    out_type=jax.ShapeDtypeStruct((N_NODES, F_OUT), jnp.float32),
    mesh=_MESH,
    scratch_types=[
        pltpu.VMEM((AGG_ROWS_PER_W, CHUNK), jnp.int32),
        pltpu.VMEM((8, CHUNK), jnp.int32),
        pltpu.VMEM((CHUNK, FH), jnp.float32),
        pltpu.VMEM((CHUNK, FH), jnp.float32),
        pltpu.VMEM_SHARED((N_NODES, FH), jnp.float32),
        pltpu.SemaphoreType.DMA,
        pltpu.SemaphoreType.DMA,
        pltpu.SemaphoreType.DMA,
        pltpu.SemaphoreType.DMA,
    ],
)
def _agg_kernel(h_hbm, e_hbm, binit_hbm, out_hbm,
                sidx_v, didx_g, rows_a, rows_b, acc_sh,
                sem_a, sem_b, sem_sa, sem_sb):
    c = lax.axis_index("c")
    s = lax.axis_index("s")

    pltpu.sync_copy(binit_hbm.at[c].at[pl.ds(s * STRIPE, STRIPE)],
                    acc_sh.at[pl.ds(s * STRIPE, STRIPE)])

    @pl.when(s == NS - 1)
    def _():
        pltpu.sync_copy(binit_hbm.at[c].at[pl.ds(NS * STRIPE, TAIL_ROWS)],
                        acc_sh.at[pl.ds(NS * STRIPE, TAIL_ROWS)])

    pltpu.sync_copy(
        e_hbm.at[0].at[pl.ds(s * AGG_ROWS_PER_W, AGG_ROWS_PER_W)],
        sidx_v)
    plsc.subcore_barrier()

    tab = h_hbm.at[c]

    pltpu.async_copy(tab.at[sidx_v.at[0]], rows_a, sem_a)
    pltpu.async_copy(tab.at[sidx_v.at[1]], rows_b, sem_b)

    @pl.loop(0, AGG_ROWS_PER_W, step=2)
    def _(j):
        @pl.when(lax.rem(j, 8) == 0)
        def _():
            base = pl.multiple_of(s * AGG_ROWS_PER_W + j, 8)
            pltpu.sync_copy(e_hbm.at[1].at[pl.ds(base, 8)], didx_g)

        k = lax.rem(j, 8)
        pltpu.make_async_copy(tab.at[sidx_v.at[j]], rows_a, sem_a).wait()
        pltpu.sync_copy(rows_a, acc_sh.at[didx_g.at[k]], add=True)

        @pl.when(j + 2 < AGG_ROWS_PER_W)
        def _():
            pltpu.async_copy(tab.at[sidx_v.at[j + 2]], rows_a, sem_a)

        pltpu.make_async_copy(tab.at[sidx_v.at[j + 1]], rows_b, sem_b).wait()
        pltpu.sync_copy(rows_b, acc_sh.at[didx_g.at[k + 1]], add=True)

        @pl.when(j + 3 < AGG_ROWS_PER_W)
        def _():
            pltpu.async_copy(tab.at[sidx_v.at[j + 3]], rows_b, sem_b)

    plsc.subcore_barrier()

    pltpu.sync_copy(
        acc_sh.at[pl.ds(s * STRIPE, STRIPE)],
        out_hbm.at[pl.ds(s * STRIPE, STRIPE), pl.ds(c * FH, FH)])

    @pl.when(s == NS - 1)
    def _():
        pltpu.sync_copy(
            acc_sh.at[pl.ds(NS * STRIPE, TAIL_ROWS)],
            out_hbm.at[pl.ds(NS * STRIPE, TAIL_ROWS), pl.ds(c * FH, FH)])


def kernel(x, edge_index, W, b):
    e3 = edge_index.astype(jnp.int32).reshape(2, E_ROWS, CHUNK)

    zeros_1d = jnp.zeros((N_NODES,), jnp.float32)
    ones_c = jnp.ones((CHUNK,), jnp.float32)
    deg0, deg1 = _deg_kernel(e3, zeros_1d, ones_c)

    h = _tc_matmul(x, W, (deg0 + deg1).reshape(N_NODES, 1))

    binit = jnp.broadcast_to(b.reshape(NC, 1, FH), (NC, N_NODES, FH))
    return _agg_kernel(h, e3, binit)


# speedup vs baseline: 1.0203x; 1.0203x over previous
import functools

import jax
import jax.numpy as jnp
from jax import lax
from jax.experimental import pallas as pl
from jax.experimental.pallas import tpu as pltpu
from jax.experimental.pallas import tpu_sc as plsc

N_NODES = 10000
N_EDGES = 160000
F_IN = 256
F_OUT = 256
FH = 128
NC = 2
NS = 16

CHUNK = 125
E_ROWS = N_EDGES // CHUNK
DEG_ROWS_PER_W = E_ROWS // (NC * NS)
AGG_ROWS_PER_W = E_ROWS // NS

STRIPE = 624
TAIL_ROWS = N_NODES - NS * STRIPE

_MESH = plsc.VectorSubcoreMesh(core_axis_name="c", subcore_axis_name="s")


@functools.partial(
    pl.kernel,
    out_type=[jax.ShapeDtypeStruct((N_NODES,), jnp.float32),
              jax.ShapeDtypeStruct((N_NODES,), jnp.float32)],
    mesh=_MESH,
    scratch_types=[
        pltpu.VMEM((DEG_ROWS_PER_W, CHUNK), jnp.int32),
        pltpu.VMEM((CHUNK,), jnp.float32),
        pltpu.VMEM_SHARED((N_NODES,), jnp.float32),
        pltpu.SemaphoreType.DMA,
    ],
)
def _deg_kernel(e_hbm, zeros_hbm, ones_hbm, deg0_hbm, deg1_hbm,
                idx_v, ones_v, deg_sh, sem):
    c = lax.axis_index("c")
    s = lax.axis_index("s")
    w = c * NS + s

    @pl.when(s == 0)
    def _():
        pltpu.sync_copy(zeros_hbm, deg_sh)

    pltpu.sync_copy(
        e_hbm.at[0].at[pl.ds(w * DEG_ROWS_PER_W, DEG_ROWS_PER_W)], idx_v)
    pltpu.sync_copy(ones_hbm, ones_v)
    plsc.subcore_barrier()

    @pl.loop(0, DEG_ROWS_PER_W, step=8)
    def _(j):
        for t in range(8):
            pltpu.async_copy(ones_v, deg_sh.at[idx_v.at[j + t]], sem,
                             add=True)
        for t in range(8):
            pltpu.make_async_copy(ones_v, deg_sh.at[idx_v.at[j + t]],
                                  sem).wait()

    plsc.subcore_barrier()

    @pl.when((s == 0) & (c == 0))
    def _():
        pltpu.sync_copy(deg_sh, deg0_hbm)

    @pl.when((s == 0) & (c == 1))
    def _():
        pltpu.sync_copy(deg_sh, deg1_hbm)


_TC_BLOCK = 2000


def _tc_body(x_ref, w_ref, deg_ref, h_ref):
    norm = lax.rsqrt(deg_ref[...] + 1.0)
    h = jnp.dot(x_ref[...].astype(jnp.bfloat16),
                w_ref[...].astype(jnp.bfloat16),
                preferred_element_type=jnp.float32)
    h = h * norm
    h_ref[0] = h[:, :FH]
    h_ref[1] = h[:, FH:]


def _tc_matmul(x, W, degsum):
    grid = (N_NODES // _TC_BLOCK,)
    return pl.pallas_call(
        _tc_body,
        grid=grid,
        in_specs=[
            pl.BlockSpec((_TC_BLOCK, F_IN), lambda i: (i, 0)),
            pl.BlockSpec((F_IN, F_OUT), lambda i: (0, 0)),
            pl.BlockSpec((_TC_BLOCK, 1), lambda i: (i, 0)),
        ],
        out_specs=pl.BlockSpec((NC, _TC_BLOCK, FH), lambda i: (0, i, 0)),
        out_shape=jax.ShapeDtypeStruct((NC, N_NODES, FH), jnp.float32),
    )(x, W, degsum)


@functools.partial(
    pl.kernel,
    out_type=jax.ShapeDtypeStruct((N_NODES, F_OUT), jnp.float32),
    mesh=_MESH,
    scratch_types=[
        pltpu.VMEM((AGG_ROWS_PER_W, CHUNK), jnp.int32),
        pltpu.VMEM((40, CHUNK), jnp.int32),
        pltpu.VMEM((CHUNK, FH), jnp.float32),
        pltpu.VMEM((CHUNK, FH), jnp.float32),
        pltpu.VMEM_SHARED((N_NODES, FH), jnp.float32),
        pltpu.SemaphoreType.DMA,
        pltpu.SemaphoreType.DMA,
        pltpu.SemaphoreType.DMA,
        pltpu.SemaphoreType.DMA,
    ],
)
def _agg_kernel(h_hbm, e_hbm, binit_hbm, out_hbm,
                sidx_v, didx_g, rows_a, rows_b, acc_sh,
                sem_a, sem_b, sem_sa, sem_sb):
    c = lax.axis_index("c")
    s = lax.axis_index("s")

    pltpu.sync_copy(binit_hbm.at[c].at[pl.ds(s * STRIPE, STRIPE)],
                    acc_sh.at[pl.ds(s * STRIPE, STRIPE)])

    @pl.when(s == NS - 1)
    def _():
        pltpu.sync_copy(binit_hbm.at[c].at[pl.ds(NS * STRIPE, TAIL_ROWS)],
                        acc_sh.at[pl.ds(NS * STRIPE, TAIL_ROWS)])

    pltpu.sync_copy(
        e_hbm.at[0].at[pl.ds(s * AGG_ROWS_PER_W, AGG_ROWS_PER_W)],
        sidx_v)
    plsc.subcore_barrier()

    tab = h_hbm.at[c]

    pltpu.async_copy(tab.at[sidx_v.at[0]], rows_a, sem_a)
    pltpu.async_copy(tab.at[sidx_v.at[1]], rows_b, sem_b)

    @pl.loop(0, AGG_ROWS_PER_W, step=2)
    def _(j):
        @pl.when(lax.rem(j, 40) == 0)
        def _():
            base = pl.multiple_of(s * AGG_ROWS_PER_W + j, 8)
            pltpu.sync_copy(e_hbm.at[1].at[pl.ds(base, 40)], didx_g)

        k = lax.rem(j, 40)
        pltpu.make_async_copy(tab.at[sidx_v.at[j]], rows_a, sem_a).wait()
        pltpu.sync_copy(rows_a, acc_sh.at[didx_g.at[k]], add=True)

        @pl.when(j + 2 < AGG_ROWS_PER_W)
        def _():
            pltpu.async_copy(tab.at[sidx_v.at[j + 2]], rows_a, sem_a)

        pltpu.make_async_copy(tab.at[sidx_v.at[j + 1]], rows_b, sem_b).wait()
        pltpu.sync_copy(rows_b, acc_sh.at[didx_g.at[k + 1]], add=True)

        @pl.when(j + 3 < AGG_ROWS_PER_W)
        def _():
            pltpu.async_copy(tab.at[sidx_v.at[j + 3]], rows_b, sem_b)

    plsc.subcore_barrier()

    pltpu.sync_copy(
        acc_sh.at[pl.ds(s * STRIPE, STRIPE)],
        out_hbm.at[pl.ds(s * STRIPE, STRIPE), pl.ds(c * FH, FH)])

    @pl.when(s == NS - 1)
    def _():
        pltpu.sync_copy(
            acc_sh.at[pl.ds(NS * STRIPE, TAIL_ROWS)],
            out_hbm.at[pl.ds(NS * STRIPE, TAIL_ROWS), pl.ds(c * FH, FH)])


def kernel(x, edge_index, W, b):
    e3 = edge_index.astype(jnp.int32).reshape(2, E_ROWS, CHUNK)

    zeros_1d = jnp.zeros((N_NODES,), jnp.float32)
    ones_c = jnp.ones((CHUNK,), jnp.float32)
    deg0, deg1 = _deg_kernel(e3, zeros_1d, ones_c)

    h = _tc_matmul(x, W, (deg0 + deg1).reshape(N_NODES, 1))

    binit = jnp.broadcast_to(b.reshape(NC, 1, FH), (NC, N_NODES, FH))
    return _agg_kernel(h, e3, binit)
